# baseline (speedup 1.0000x reference)
import jax
import jax.numpy as jnp
from jax.experimental import pallas as pl
from jax.experimental.pallas import tpu as pltpu

_B, _S, _D, _H, _E, _K = 32, 4, 768, 256, 8, 2
_EPS_NORM = 1e-6


def _moe_kernel(x_ref, nw_ref, gwt_ref, w1_hbm, w2_hbm, w3_hbm, out_ref,
                w1_buf, w2_buf, w3_buf, sems):
    def _start(e, slot):
        pltpu.make_async_copy(w1_hbm.at[e], w1_buf.at[slot], sems.at[slot, 0]).start()
        pltpu.make_async_copy(w2_hbm.at[e], w2_buf.at[slot], sems.at[slot, 1]).start()
        pltpu.make_async_copy(w3_hbm.at[e], w3_buf.at[slot], sems.at[slot, 2]).start()

    def _wait(e, slot):
        pltpu.make_async_copy(w1_hbm.at[e], w1_buf.at[slot], sems.at[slot, 0]).wait()
        pltpu.make_async_copy(w2_hbm.at[e], w2_buf.at[slot], sems.at[slot, 1]).wait()
        pltpu.make_async_copy(w3_hbm.at[e], w3_buf.at[slot], sems.at[slot, 2]).wait()

    _start(0, 0)
    _start(1, 1)

    x = x_ref[...]
    nw = nw_ref[...]
    xn = x * jax.lax.rsqrt(jnp.mean(x * x, axis=-1, keepdims=True) + _EPS_NORM)
    xn = xn * nw

    logits = jnp.dot(xn, gwt_ref[...], preferred_element_type=jnp.float32)
    p = jax.nn.softmax(logits, axis=-1)
    iota = jax.lax.broadcasted_iota(jnp.int32, p.shape, 1)
    m1 = jnp.max(p, axis=-1, keepdims=True)
    i1 = jnp.min(jnp.where(p >= m1, iota, _E), axis=-1, keepdims=True)
    one1 = iota == i1
    p2 = jnp.where(one1, -1.0, p)
    m2 = jnp.max(p2, axis=-1, keepdims=True)
    i2 = jnp.min(jnp.where(p2 >= m2, iota, _E), axis=-1, keepdims=True)
    one2 = iota == i2
    c = jnp.where(one1 | one2, p, 0.0) / (m1 + m2 + 1e-10)

    acc = jnp.zeros(out_ref.shape, jnp.float32)
    for e in range(_E):
        slot = e % 2
        _wait(e, slot)
        h1 = jnp.dot(xn, w1_buf[slot], preferred_element_type=jnp.float32)
        h2 = jnp.dot(xn, w2_buf[slot], preferred_element_type=jnp.float32)
        hid = (h1 * jax.lax.logistic(h1)) * h2
        oe = jnp.dot(hid, w3_buf[slot], preferred_element_type=jnp.float32)
        acc = acc + c[:, e:e + 1] * oe
        if e + 2 < _E:
            _start(e + 2, slot)
    out_ref[...] = acc


def kernel(x, norm_weight, gate_w, w1, w2, w3):
    b, s, d = x.shape
    t = b * s
    x_flat = x.reshape(t, d)
    nw = norm_weight.reshape(1, d)
    gwt = gate_w.T
    out = pl.pallas_call(
        _moe_kernel,
        in_specs=[
            pl.BlockSpec((t, d), lambda: (0, 0)),
            pl.BlockSpec((1, d), lambda: (0, 0)),
            pl.BlockSpec((d, _E), lambda: (0, 0)),
            pl.BlockSpec(memory_space=pl.ANY),
            pl.BlockSpec(memory_space=pl.ANY),
            pl.BlockSpec(memory_space=pl.ANY),
        ],
        out_specs=pl.BlockSpec((t, d), lambda: (0, 0)),
        out_shape=jax.ShapeDtypeStruct((t, d), jnp.float32),
        scratch_shapes=[
            pltpu.VMEM((2, _D, _H), jnp.float32),
            pltpu.VMEM((2, _D, _H), jnp.float32),
            pltpu.VMEM((2, _H, _D), jnp.float32),
            pltpu.SemaphoreType.DMA((2, 3)),
        ],
    )(x_flat, nw, gwt, w1, w2, w3)
    return out.reshape(b, s, d)


# speedup vs baseline: 1.0743x; 1.0743x over previous
import jax
import jax.numpy as jnp
from jax.experimental import pallas as pl
from jax.experimental.pallas import tpu as pltpu

_B, _S, _D, _H, _E, _K = 32, 4, 768, 256, 8, 2
_EPS_NORM = 1e-6


def _moe_kernel(x_ref, nw_ref, gwt_ref, w1_hbm, w2_hbm, w3_hbm, out_ref,
                w1_buf, w2_buf, w3_buf, sems):
    def _copies(e):
        return (
            pltpu.make_async_copy(w1_hbm.at[e], w1_buf.at[e], sems.at[e, 0]),
            pltpu.make_async_copy(w2_hbm.at[e], w2_buf.at[e], sems.at[e, 1]),
            pltpu.make_async_copy(w3_hbm.at[e], w3_buf.at[e], sems.at[e, 2]),
        )

    for e in range(_E):
        for cp in _copies(e):
            cp.start()

    x = x_ref[...]
    nw = nw_ref[...]
    xn = x * jax.lax.rsqrt(jnp.mean(x * x, axis=-1, keepdims=True) + _EPS_NORM)
    xn = xn * nw

    logits = jnp.dot(xn, gwt_ref[...], preferred_element_type=jnp.float32)
    p = jax.nn.softmax(logits, axis=-1)
    iota = jax.lax.broadcasted_iota(jnp.int32, p.shape, 1)
    m1 = jnp.max(p, axis=-1, keepdims=True)
    i1 = jnp.min(jnp.where(p >= m1, iota, _E), axis=-1, keepdims=True)
    one1 = iota == i1
    p2 = jnp.where(one1, -1.0, p)
    m2 = jnp.max(p2, axis=-1, keepdims=True)
    i2 = jnp.min(jnp.where(p2 >= m2, iota, _E), axis=-1, keepdims=True)
    one2 = iota == i2
    c = jnp.where(one1 | one2, p, 0.0) / (m1 + m2 + 1e-10)

    acc = jnp.zeros(out_ref.shape, jnp.float32)
    for e in range(_E):
        for cp in _copies(e):
            cp.wait()
        h1 = jnp.dot(xn, w1_buf[e], preferred_element_type=jnp.float32)
        h2 = jnp.dot(xn, w2_buf[e], preferred_element_type=jnp.float32)
        hid = (h1 * jax.lax.logistic(h1)) * h2
        oe = jnp.dot(hid, w3_buf[e], preferred_element_type=jnp.float32)
        acc = acc + c[:, e:e + 1] * oe
    out_ref[...] = acc


def kernel(x, norm_weight, gate_w, w1, w2, w3):
    b, s, d = x.shape
    t = b * s
    x_flat = x.reshape(t, d)
    nw = norm_weight.reshape(1, d)
    gwt = gate_w.T
    out = pl.pallas_call(
        _moe_kernel,
        in_specs=[
            pl.BlockSpec((t, d), lambda: (0, 0)),
            pl.BlockSpec((1, d), lambda: (0, 0)),
            pl.BlockSpec((d, _E), lambda: (0, 0)),
            pl.BlockSpec(memory_space=pl.ANY),
            pl.BlockSpec(memory_space=pl.ANY),
            pl.BlockSpec(memory_space=pl.ANY),
        ],
        out_specs=pl.BlockSpec((t, d), lambda: (0, 0)),
        out_shape=jax.ShapeDtypeStruct((t, d), jnp.float32),
        scratch_shapes=[
            pltpu.VMEM((_E, _D, _H), jnp.float32),
            pltpu.VMEM((_E, _D, _H), jnp.float32),
            pltpu.VMEM((_E, _H, _D), jnp.float32),
            pltpu.SemaphoreType.DMA((_E, 3)),
        ],
    )(x_flat, nw, gwt, w1, w2, w3)
    return out.reshape(b, s, d)
